# Initial kernel scaffold; baseline (speedup 1.0000x reference)
import jax
import jax.numpy as jnp
from jax.experimental import pallas as pl


def kernel(q, kv_cache, W_cmp_kv, W_cmp_kpe, W_kvb):
    raise NotImplementedError("write your pallas kernel here")


# speedup vs baseline: 1.1701x; 1.1701x over previous
import numpy as np
import jax
import jax.numpy as jnp
from jax.experimental import pallas as pl
from jax.experimental.pallas import tpu as pltpu

T = 4096
H = 16
D_LORA = 512
D_ROPE = 64
D_NOPE = 128
D_V = 128
D_QK = D_NOPE + D_ROPE
KS = 32
ST = 16
SEL = 64
TOPK = 16
INIT_B = 1
LOCAL_B = 2
SCALE = D_QK ** -0.5
NB = (T - KS) // ST + 1
NBP = 256
NSEL = (T + SEL - 1) // SEL
NCH = T // ST

TQ = 256
KK = 2
CK = (KS * D_LORA) // KK


def _overlap_np():
    ov = np.zeros((NBP, NSEL), dtype=np.float32)
    for i in range(NB):
        s0 = i * ST
        s1 = s0 + KS
        for j in range(NSEL):
            t0 = j * SEL
            t1 = t0 + SEL
            ov[i, j] = max(0, min(s1, t1) - max(s0, t0)) / float(KS)
    return ov


_OV = _overlap_np()


def _compress_body(x_ref, xpe_ref, w_ref, wpe_ref, cmp_ref, kpe_ref, acc_ref):
    kk = pl.program_id(0)

    @pl.when(kk == 0)
    def _init():
        acc_ref[...] = jnp.zeros_like(acc_ref)
        kpe_ref[...] = jnp.dot(xpe_ref[...], wpe_ref[...],
                               preferred_element_type=jnp.float32)

    acc_ref[...] += jnp.dot(x_ref[...], w_ref[...],
                            preferred_element_type=jnp.float32)

    @pl.when(kk == pl.num_programs(0) - 1)
    def _fin():
        cmp_ref[...] = acc_ref[...]


def _kvb_body(cmp_ref, kpe_ref, wkvb_ref, k3_ref, v3_ref):
    kvb = jnp.dot(cmp_ref[...], wkvb_ref[...],
                  preferred_element_type=jnp.float32)
    kpe = kpe_ref[...]
    for h in range(H):
        base = h * (D_NOPE + D_V)
        k3_ref[h] = jnp.concatenate(
            [kvb[:, base:base + D_NOPE], kpe], axis=1)
        v3_ref[h] = kvb[:, base + D_NOPE:base + D_NOPE + D_V]


def _attn_body(q_ref, k_ref, v_ref, ov_ref, o_ref, idx_ref):
    t0 = pl.program_id(0) * TQ
    tpos = t0 + jax.lax.broadcasted_iota(jnp.int32, (TQ, NBP), 0)
    blk_last = jax.lax.broadcasted_iota(jnp.int32, (TQ, NBP), 1) * ST + (KS - 1)
    vis = tpos >= blk_last
    anyv = (tpos[:, :1] >= (KS - 1)).astype(jnp.float32)

    ps = []
    for h in range(H):
        q_t = q_ref[:, h, :]
        k_h = k_ref[h]
        s = jax.lax.dot_general(
            q_t, k_h, (((1,), (1,)), ((), ())),
            preferred_element_type=jnp.float32) * SCALE
        s = jnp.where(vis, s, -1e9)
        m = jnp.max(s, axis=1, keepdims=True)
        e = jnp.exp(s - m)
        den = jnp.sum(e[:, :128] + e[:, 128:], axis=1, keepdims=True)
        p = e / den
        p = p * anyv
        ps.append(p)
        o_ref[:, h * D_V:(h + 1) * D_V] = jnp.dot(
            p, v_ref[h], preferred_element_type=jnp.float32)

    lvl = ps
    while len(lvl) > 1:
        lvl = [lvl[i] + lvl[i + 1] for i in range(0, len(lvl), 2)]
    psum = lvl[0]

    sel = jnp.dot(psum, ov_ref[...], preferred_element_type=jnp.float32)
    rows = t0 + jax.lax.broadcasted_iota(jnp.int32, (TQ, NSEL), 0)
    cur = rows // SEL
    j = jax.lax.broadcasted_iota(jnp.int32, (TQ, NSEL), 1)
    causal = j <= cur
    dist = cur - j
    forced = (j < INIT_B) | ((dist >= 0) & (dist < LOCAL_B))
    sel = jnp.where(causal, sel, -1e9)
    sel = jnp.where(forced & causal, 1e9, sel)

    vals = sel
    iota_n = j
    idx_mat = jnp.zeros((TQ, TOPK), jnp.int32)
    iota_k = jax.lax.broadcasted_iota(jnp.int32, (TQ, TOPK), 1)
    for i in range(TOPK):
        mx = jnp.max(vals, axis=1, keepdims=True)
        idx = jnp.min(jnp.where(vals == mx, iota_n, NSEL), axis=1)
        idx_mat = jnp.where(iota_k == i, idx[:, None], idx_mat)
        vals = jnp.where(iota_n == idx[:, None], jnp.float32(-jnp.inf), vals)
    idx_ref[...] = idx_mat


def kernel(q, kv_cache, W_cmp_kv, W_cmp_kpe, W_kvb):
    ckv = kv_cache[:, :D_LORA].reshape(NCH, ST * D_LORA)
    cpe = kv_cache[:, D_LORA:].reshape(NCH, ST * D_ROPE)
    ckv_n = jnp.concatenate([ckv[1:], jnp.zeros((1, ST * D_LORA), ckv.dtype)], 0)
    cpe_n = jnp.concatenate([cpe[1:], jnp.zeros((1, ST * D_ROPE), cpe.dtype)], 0)
    X = jnp.concatenate([ckv, ckv_n], axis=1)
    Xpe = jnp.concatenate([cpe, cpe_n], axis=1)

    cmp, kpe = pl.pallas_call(
        _compress_body,
        grid=(KK,),
        in_specs=[
            pl.BlockSpec((NCH, CK), lambda kk: (0, kk)),
            pl.BlockSpec((NCH, KS * D_ROPE), lambda kk: (0, 0)),
            pl.BlockSpec((CK, D_LORA), lambda kk: (kk, 0)),
            pl.BlockSpec((KS * D_ROPE, D_ROPE), lambda kk: (0, 0)),
        ],
        out_specs=[
            pl.BlockSpec((NCH, D_LORA), lambda kk: (0, 0)),
            pl.BlockSpec((NCH, D_ROPE), lambda kk: (0, 0)),
        ],
        out_shape=[
            jax.ShapeDtypeStruct((NCH, D_LORA), jnp.float32),
            jax.ShapeDtypeStruct((NCH, D_ROPE), jnp.float32),
        ],
        scratch_shapes=[
            pltpu.VMEM((NCH, D_LORA), jnp.float32),
        ],
    )(X, Xpe, W_cmp_kv, W_cmp_kpe)

    k3, v3 = pl.pallas_call(
        _kvb_body,
        out_shape=[
            jax.ShapeDtypeStruct((H, NBP, D_QK), jnp.float32),
            jax.ShapeDtypeStruct((H, NBP, D_V), jnp.float32),
        ],
    )(cmp, kpe, W_kvb)

    ov = jnp.asarray(_OV)
    o_flat, idx = pl.pallas_call(
        _attn_body,
        grid=(T // TQ,),
        in_specs=[
            pl.BlockSpec((TQ, H, D_QK), lambda t: (t, 0, 0)),
            pl.BlockSpec((H, NBP, D_QK), lambda t: (0, 0, 0)),
            pl.BlockSpec((H, NBP, D_V), lambda t: (0, 0, 0)),
            pl.BlockSpec((NBP, NSEL), lambda t: (0, 0)),
        ],
        out_specs=[
            pl.BlockSpec((TQ, H * D_V), lambda t: (t, 0)),
            pl.BlockSpec((TQ, TOPK), lambda t: (t, 0)),
        ],
        out_shape=[
            jax.ShapeDtypeStruct((T, H * D_V), jnp.float32),
            jax.ShapeDtypeStruct((T, TOPK), jnp.int32),
        ],
    )(q, k3, v3, ov)

    return o_flat, idx.reshape(T, 1, TOPK)
